# Initial kernel scaffold; baseline (speedup 1.0000x reference)
import jax
import jax.numpy as jnp
from jax.experimental import pallas as pl


def kernel(inputs, targets):
    raise NotImplementedError("write your pallas kernel here")


# speedup vs baseline: 3.3503x; 3.3503x over previous
import functools

import jax
import jax.numpy as jnp
from jax.experimental import pallas as pl
from jax.experimental.pallas import tpu as pltpu

_N = 8192
_C = 1000
_BLOCK = 512
_GAMMA = 2.0


def _loss_kernel(x_ref, t_ref, out_ref, counts_ref, focal_ref):
    step = pl.program_id(0)
    nsteps = pl.num_programs(0)

    @pl.when(step == 0)
    def _init():
        counts_ref[...] = jnp.zeros_like(counts_ref)
        focal_ref[0, 0] = 0.0

    x = x_ref[...]
    t = t_ref[...]
    m = jnp.max(x, axis=1, keepdims=True)
    z = jnp.sum(jnp.exp(x - m), axis=1, keepdims=True)
    col = jax.lax.broadcasted_iota(jnp.int32, x.shape, 1)
    mask = col == t
    xt = jnp.sum(jnp.where(mask, x, 0.0), axis=1, keepdims=True)
    logp = xt - m - jnp.log(z)
    p = jnp.exp(logp)
    one_m_p = 1.0 - p
    focal_block = jnp.sum(one_m_p * one_m_p * (-logp))
    counts_ref[...] += jnp.sum(mask.astype(jnp.float32), axis=0,
                               keepdims=True)
    focal_ref[0, 0] += focal_block

    @pl.when(step == nsteps - 1)
    def _finish():
        counts = counts_ref[...]
        n = jnp.float32(_N)
        s_alpha = n - jnp.sum(counts * counts) / (10.0 * n)
        loss = s_alpha * focal_ref[0, 0] / (n * n)
        out_ref[...] = jnp.full((1, 1), loss, dtype=jnp.float32)


@jax.jit
def kernel(inputs, targets):
    grid = _N // _BLOCK
    t2d = targets.reshape(_N, 1)
    out = pl.pallas_call(
        _loss_kernel,
        grid=(grid,),
        in_specs=[
            pl.BlockSpec((_BLOCK, _C), lambda i: (i, 0)),
            pl.BlockSpec((_BLOCK, 1), lambda i: (i, 0)),
        ],
        out_specs=pl.BlockSpec((1, 1), lambda i: (0, 0)),
        out_shape=jax.ShapeDtypeStruct((1, 1), jnp.float32),
        scratch_shapes=[
            pltpu.VMEM((1, _C), jnp.float32),
            pltpu.SMEM((1, 1), jnp.float32),
        ],
    )(inputs, t2d)
    return out[0, 0]
